# Initial kernel scaffold; baseline (speedup 1.0000x reference)
import jax
import jax.numpy as jnp
from jax.experimental import pallas as pl


def kernel(x_omic, emb_var, emb_vc, emb_func, W, b):
    raise NotImplementedError("write your pallas kernel here")


# speedup vs baseline: 5.8247x; 5.8247x over previous
import functools

import jax
import jax.numpy as jnp
from jax import lax
from jax.experimental import pallas as pl
from jax.experimental.pallas import tpu as pltpu
from jax.experimental.pallas import tpu_sc as plsc

_B, _L, _OUT = 128, 1425, 256
_N = _B * _L

_NC, _NS = 2, 16
_NW = _NC * _NS
_CHUNK = 128
_CPW = -(-_N // (_NW * _CHUNK))
_NPAD = _NW * _CPW * _CHUNK

_T = 1600
_STEPS = _N // _T


def _sc_gather_body(table_hbm, idx_hbm, out_hbm, idx_v, rows_v, sem):
    wid = lax.axis_index("s") * _NC + lax.axis_index("c")
    base = wid * _CPW * _CHUNK
    pltpu.sync_copy(idx_hbm.at[pl.ds(base, _CPW * _CHUNK)], idx_v)

    def body(g, carry):
        iref = idx_v.at[pl.ds(g * _CHUNK, _CHUNK)]
        pltpu.async_copy(table_hbm.at[iref], rows_v, sem).wait()
        pltpu.sync_copy(rows_v, out_hbm.at[pl.ds(base + g * _CHUNK, _CHUNK)])
        return carry

    lax.fori_loop(0, _CPW, body, 0)


def _sc_gather(table, idx2):
    mesh = plsc.VectorSubcoreMesh(core_axis_name="c", subcore_axis_name="s")
    fn = pl.kernel(
        _sc_gather_body,
        out_type=jax.ShapeDtypeStruct((_NPAD, 128), jnp.float32),
        mesh=mesh,
        scratch_types=[
            pltpu.VMEM((_CPW * _CHUNK,), jnp.int32),
            pltpu.VMEM((_CHUNK, 128), jnp.float32),
            pltpu.SemaphoreType.DMA,
        ],
    )
    return fn(table, idx2)


def _tc_body(x_ref, hv_ref, evc_ref, efn_ref, w_ref, b_ref, o_ref, wf_ref):
    @pl.when(pl.program_id(0) == 0)
    def _():
        wvc = jnp.dot(evc_ref[...], w_ref[128:160, :],
                      preferred_element_type=jnp.float32)
        wfn = jnp.dot(efn_ref[...], w_ref[160:192, :],
                      preferred_element_type=jnp.float32)
        z7 = jnp.zeros((7, 256), jnp.float32)
        z15 = jnp.zeros((15, 256), jnp.float32)
        wf_ref[...] = jnp.concatenate(
            [w_ref[0:128, :], wvc, z7, wfn, z7, w_ref[192:193, :], z15], axis=0)

    x = x_ref[...]
    hv = hv_ref[...]
    iota = lax.broadcasted_iota(jnp.int32, (_T, 128), 1).astype(jnp.float32)
    a = (x[:, 1:2] == iota).astype(jnp.float32)
    sixth = jnp.float32(1.0 / 6.0)
    for k in range(6):
        a = a + jnp.where(x[:, 2 + k:3 + k] == iota - 40.0, sixth, 0.0)
    a = a + jnp.where(iota == 112.0, x[:, 8:9], 0.0)
    h2 = jnp.concatenate([hv, a], axis=1)
    y = jnp.dot(h2, wf_ref[...], preferred_element_type=jnp.float32) + b_ref[...]
    o_ref[...] = jnp.where(y > 0.0, y, jnp.exp(jnp.minimum(y, 0.0)) - 1.0)


def _tc_call(x2, hvar, emb_vc, emb_func, w, b):
    return pl.pallas_call(
        _tc_body,
        grid=(_STEPS,),
        in_specs=[
            pl.BlockSpec((_T, 9), lambda i: (i, 0)),
            pl.BlockSpec((_T, 128), lambda i: (i, 0)),
            pl.BlockSpec((33, 32), lambda i: (0, 0)),
            pl.BlockSpec((65, 32), lambda i: (0, 0)),
            pl.BlockSpec((193, 256), lambda i: (0, 0)),
            pl.BlockSpec((1, 256), lambda i: (0, 0)),
        ],
        out_specs=pl.BlockSpec((_T, 256), lambda i: (i, 0)),
        out_shape=jax.ShapeDtypeStruct((_N, 256), jnp.float32),
        scratch_shapes=[pltpu.VMEM((256, 256), jnp.float32)],
    )(x2, hvar, emb_vc, emb_func, w, b)


def kernel(x_omic, emb_var, emb_vc, emb_func, W, b):
    x2 = x_omic.reshape(_N, 9)
    var_idx = x2[:, 0].astype(jnp.int32)
    idx1 = jnp.pad(var_idx, (0, _NPAD - _N))
    hvar = _sc_gather(emb_var, idx1)
    out = _tc_call(x2, hvar, emb_vc, emb_func, W, b.reshape(1, _OUT))
    return out.reshape(_B, _L, _OUT)
